# baseline (speedup 1.0000x reference)
import functools

import jax
import jax.numpy as jnp
from jax import lax
from jax.experimental import pallas as pl
from jax.experimental.pallas import tpu as pltpu
from jax.experimental.pallas import tpu_sc as plsc

N_NODES = 10000
DEG = 32
D = 128
N_CLASSES = 104
N_EDGES = N_NODES * DEG
VOCAB = 50000

NW = 32
NS = 16
NB = 320
P = NW * NB
CH = 4
NCHUNK = NB // CH
EPC = CH * DEG
NBUF = 2
TPW = P // NS
TCH = TPW // EPC
D2 = D // 2

_mesh = plsc.VectorSubcoreMesh(core_axis_name="c", subcore_axis_name="s")


@functools.partial(
    pl.kernel,
    mesh=_mesh,
    compiler_params=pltpu.CompilerParams(use_tc_tiling_on_sc=False),
    out_type=jax.ShapeDtypeStruct((P, 2, D), jnp.float32),
    scratch_types=[
        pltpu.VMEM((NCHUNK, EPC), jnp.int32),
        pltpu.VMEM((TCH, EPC), jnp.int32),
        pltpu.VMEM((NBUF, EPC, D2), jnp.int32),
        pltpu.VMEM((2, CH, 2, D), jnp.float32),
        pltpu.VMEM_SHARED((P, D2), jnp.int32),
        pltpu.SemaphoreType.DMA,
        pltpu.SemaphoreType.DMA,
        pltpu.SemaphoreType.DMA,
        pltpu.SemaphoreType.DMA,
        pltpu.SemaphoreType.DMA,
    ],
)
def _sc_gather_reduce(edge_hbm, tok_hbm, emb_hbm, tl_hbm,
                      edge_v, tok_v, rows_v, tl_v, feat_sp,
                      sem_t, sem_r0, sem_r1,
                      sem_w0, sem_w1):
    sem_r = [sem_r0, sem_r1]
    sem_w = [sem_w0, sem_w1]
    cid = lax.axis_index("c")
    sid = lax.axis_index("s")
    wid = sid * 2 + cid
    pltpu.sync_copy(edge_hbm.at[wid], edge_v)

    pltpu.sync_copy(tok_hbm.at[sid], tok_v)

    def stage_fire(k, i):
        pltpu.async_copy(emb_hbm.at[tok_v.at[k]], rows_v.at[i], sem_t)

    def stage_wait(i):
        pltpu.make_async_copy(emb_hbm.at[tok_v.at[0]], rows_v.at[i],
                              sem_t).wait()

    stage_fire(0, 0)
    for k in range(TCH):
        i = k % NBUF
        if k + 1 < TCH:
            stage_fire(k + 1, (k + 1) % NBUF)
        stage_wait(i)
        pltpu.sync_copy(rows_v.at[i],
                        feat_sp.at[pl.ds(sid * TPW + k * EPC, EPC)])
    plsc.subcore_barrier()

    base_node = wid * NB

    def fire_row(c, i):
        pltpu.async_copy(feat_sp.at[edge_v.at[c]], rows_v.at[i], sem_r[i])

    def wait_row(i):
        pltpu.make_async_copy(feat_sp.at[edge_v.at[0]], rows_v.at[i],
                              sem_r[i]).wait()

    def wait_write(i):
        pltpu.make_async_copy(tl_v.at[i], tl_hbm.at[pl.ds(0, CH)],
                              sem_w[i]).wait()

    def reduce_chunk(i, j):
        rows = rows_v.at[i]
        tl = tl_v.at[j]
        for n in range(CH):
            for d in range(D2 // 16):
                sl = pl.ds(d * 16, 16)

                def load_pair(r):
                    w = rows[n * DEG + r, sl]
                    a = lax.bitcast_convert_type(
                        jnp.left_shift(w, jnp.int32(16)), jnp.float32)
                    b = lax.bitcast_convert_type(
                        w & jnp.int32(-0x10000), jnp.float32)
                    return a, b

                acc_a, acc_b = load_pair(0)
                for r in range(1, DEG):
                    a, b = load_pair(r)
                    acc_a = acc_a + a
                    acc_b = acc_b + b
                tl[n, 0, sl] = acc_a
                tl[n, 0, pl.ds(64 + d * 16, 16)] = acc_b
                tl[n, 1, sl] = a
                tl[n, 1, pl.ds(64 + d * 16, 16)] = b

    for i in range(NBUF):
        fire_row(i, i)

    def body(t, carry):
        for i in range(NBUF):
            c = NBUF * t + i
            wait_row(i)

            @pl.when(t > 0)
            def _():
                wait_write(i)

            reduce_chunk(i, i)
            pltpu.async_copy(tl_v.at[i],
                             tl_hbm.at[pl.ds(base_node + c * CH, CH)],
                             sem_w[i])

            @pl.when(c + NBUF < NCHUNK)
            def _():
                fire_row(c + NBUF, i)
        return carry

    lax.fori_loop(0, NCHUNK // NBUF, body, 0)
    for i in range(NBUF):
        wait_write(i)


VBLK = 5000


def _tc_pack_body(emb_ref, out_ref):
    u = lax.bitcast_convert_type(emb_ref[...], jnp.uint32)
    a = u[:, :D2]
    b = u[:, D2:]
    ar = jnp.right_shift(a + jnp.uint32(0x8000), jnp.uint32(16))
    br = (b + jnp.uint32(0x8000)) & jnp.uint32(0xFFFF0000)
    out_ref[...] = lax.bitcast_convert_type(ar | br, jnp.int32)


def _tc_pack(emb):
    return pl.pallas_call(
        _tc_pack_body,
        grid=(VOCAB // VBLK,),
        in_specs=[pl.BlockSpec((VBLK, D), lambda i: (i, 0))],
        out_specs=pl.BlockSpec((VBLK, D2), lambda i: (i, 0)),
        out_shape=jax.ShapeDtypeStruct((VOCAB, D2), jnp.int32),
    )(emb)


BLK = P // 8


def _tc_head_body(tl_ref, w_ref, b_ref, fc_ref, fcb_ref, out_ref):
    tot = tl_ref[:, 0, :]
    last = tl_ref[:, 1, :]
    pre = lax.dot_general(tot - last, w_ref[...], (((1,), (1,)), ((), ())),
                          preferred_element_type=jnp.float32)
    h = last + jnp.maximum(pre + b_ref[...], 0.0)
    out = lax.dot_general(h, fc_ref[...], (((1,), (1,)), ((), ())),
                          preferred_element_type=jnp.float32)
    out_ref[...] = out + fcb_ref[...]


def _tc_head(tl, W_weight, W_bias, fc_weight, fc_bias):
    return pl.pallas_call(
        _tc_head_body,
        grid=(P // BLK,),
        in_specs=[
            pl.BlockSpec((BLK, 2, D), lambda i: (i, 0, 0)),
            pl.BlockSpec((D, D), lambda i: (0, 0)),
            pl.BlockSpec((1, D), lambda i: (0, 0)),
            pl.BlockSpec((N_CLASSES, D), lambda i: (0, 0)),
            pl.BlockSpec((1, N_CLASSES), lambda i: (0, 0)),
        ],
        out_specs=pl.BlockSpec((BLK, N_CLASSES), lambda i: (i, 0)),
        out_shape=jax.ShapeDtypeStruct((P, N_CLASSES), jnp.float32),
    )(tl, W_weight, W_bias.reshape(1, D),
      fc_weight, fc_bias.reshape(1, N_CLASSES))


def kernel(token_ids, edge_src, emb_table, W_weight, W_bias, fc_weight, fc_bias):
    packed_emb = _tc_pack(emb_table)
    tok = token_ids.astype(jnp.int32)
    tok_p = jnp.pad(tok, (0, P - N_NODES)).reshape(NS, TCH, EPC)
    es = edge_src.astype(jnp.int32)
    es_p = jnp.pad(es, (0, P * DEG - N_EDGES)).reshape(NW, NCHUNK, EPC)
    tl = _sc_gather_reduce(es_p, tok_p, packed_emb)
    out = _tc_head(tl, W_weight, W_bias, fc_weight, fc_bias)
    return out[:N_NODES]


# speedup vs baseline: 1.0177x; 1.0177x over previous
import functools

import jax
import jax.numpy as jnp
from jax import lax
from jax.experimental import pallas as pl
from jax.experimental.pallas import tpu as pltpu
from jax.experimental.pallas import tpu_sc as plsc

N_NODES = 10000
DEG = 32
D = 128
N_CLASSES = 104
N_EDGES = N_NODES * DEG
VOCAB = 50000

NW = 32
NS = 16
NB = 320
P = NW * NB
CH = 2
NCHUNK = NB // CH
EPC = CH * DEG
NBUF = 2
TPW = P // NS
TCH = TPW // EPC
D2 = D // 2

_mesh = plsc.VectorSubcoreMesh(core_axis_name="c", subcore_axis_name="s")


@functools.partial(
    pl.kernel,
    mesh=_mesh,
    compiler_params=pltpu.CompilerParams(use_tc_tiling_on_sc=False),
    out_type=jax.ShapeDtypeStruct((P, 2, D), jnp.float32),
    scratch_types=[
        pltpu.VMEM((NCHUNK, EPC), jnp.int32),
        pltpu.VMEM((TCH, EPC), jnp.int32),
        pltpu.VMEM((NBUF, EPC, D2), jnp.int32),
        pltpu.VMEM((2, CH, 2, D), jnp.float32),
        pltpu.VMEM_SHARED((P, D2), jnp.int32),
        pltpu.SemaphoreType.DMA,
        pltpu.SemaphoreType.DMA,
        pltpu.SemaphoreType.DMA,
        pltpu.SemaphoreType.DMA,
        pltpu.SemaphoreType.DMA,
    ],
)
def _sc_gather_reduce(edge_hbm, tok_hbm, emb_hbm, tl_hbm,
                      edge_v, tok_v, rows_v, tl_v, feat_sp,
                      sem_t, sem_r0, sem_r1,
                      sem_w0, sem_w1):
    sem_r = [sem_r0, sem_r1]
    sem_w = [sem_w0, sem_w1]
    cid = lax.axis_index("c")
    sid = lax.axis_index("s")
    wid = sid * 2 + cid
    pltpu.sync_copy(edge_hbm.at[wid], edge_v)

    pltpu.sync_copy(tok_hbm.at[sid], tok_v)

    def stage_fire(k, i):
        pltpu.async_copy(emb_hbm.at[tok_v.at[k]], rows_v.at[i], sem_t)

    def stage_wait(i):
        pltpu.make_async_copy(emb_hbm.at[tok_v.at[0]], rows_v.at[i],
                              sem_t).wait()

    stage_fire(0, 0)
    for k in range(TCH):
        i = k % NBUF
        if k + 1 < TCH:
            stage_fire(k + 1, (k + 1) % NBUF)
        stage_wait(i)
        pltpu.sync_copy(rows_v.at[i],
                        feat_sp.at[pl.ds(sid * TPW + k * EPC, EPC)])
    plsc.subcore_barrier()

    base_node = wid * NB

    def fire_row(c, i):
        pltpu.async_copy(feat_sp.at[edge_v.at[c]], rows_v.at[i], sem_r[i])

    def wait_row(i):
        pltpu.make_async_copy(feat_sp.at[edge_v.at[0]], rows_v.at[i],
                              sem_r[i]).wait()

    def wait_write(i):
        pltpu.make_async_copy(tl_v.at[i], tl_hbm.at[pl.ds(0, CH)],
                              sem_w[i]).wait()

    def reduce_chunk(i, j):
        rows = rows_v.at[i]
        tl = tl_v.at[j]
        for n in range(CH):
            for d in range(D2 // 16):
                sl = pl.ds(d * 16, 16)

                def load_pair(r):
                    w = rows[n * DEG + r, sl]
                    a = lax.bitcast_convert_type(
                        jnp.left_shift(w, jnp.int32(16)), jnp.float32)
                    b = lax.bitcast_convert_type(
                        w & jnp.int32(-0x10000), jnp.float32)
                    return a, b

                acc_a, acc_b = load_pair(0)
                for r in range(1, DEG):
                    a, b = load_pair(r)
                    acc_a = acc_a + a
                    acc_b = acc_b + b
                tl[n, 0, sl] = acc_a
                tl[n, 0, pl.ds(64 + d * 16, 16)] = acc_b
                tl[n, 1, sl] = a
                tl[n, 1, pl.ds(64 + d * 16, 16)] = b

    for i in range(NBUF):
        fire_row(i, i)

    def body(t, carry):
        for i in range(NBUF):
            c = NBUF * t + i
            wait_row(i)

            @pl.when(t > 0)
            def _():
                wait_write(i)

            reduce_chunk(i, i)
            pltpu.async_copy(tl_v.at[i],
                             tl_hbm.at[pl.ds(base_node + c * CH, CH)],
                             sem_w[i])

            @pl.when(c + NBUF < NCHUNK)
            def _():
                fire_row(c + NBUF, i)
        return carry

    lax.fori_loop(0, NCHUNK // NBUF, body, 0)
    for i in range(NBUF):
        wait_write(i)


VBLK = 5000


def _tc_pack_body(emb_ref, out_ref):
    u = lax.bitcast_convert_type(emb_ref[...], jnp.uint32)
    a = u[:, :D2]
    b = u[:, D2:]
    ar = jnp.right_shift(a + jnp.uint32(0x8000), jnp.uint32(16))
    br = (b + jnp.uint32(0x8000)) & jnp.uint32(0xFFFF0000)
    out_ref[...] = lax.bitcast_convert_type(ar | br, jnp.int32)


def _tc_pack(emb):
    return pl.pallas_call(
        _tc_pack_body,
        grid=(VOCAB // VBLK,),
        in_specs=[pl.BlockSpec((VBLK, D), lambda i: (i, 0))],
        out_specs=pl.BlockSpec((VBLK, D2), lambda i: (i, 0)),
        out_shape=jax.ShapeDtypeStruct((VOCAB, D2), jnp.int32),
    )(emb)


BLK = P // 8


def _tc_head_body(tl_ref, w_ref, b_ref, fc_ref, fcb_ref, out_ref):
    tot = tl_ref[:, 0, :]
    last = tl_ref[:, 1, :]
    pre = lax.dot_general(tot - last, w_ref[...], (((1,), (1,)), ((), ())),
                          preferred_element_type=jnp.float32)
    h = last + jnp.maximum(pre + b_ref[...], 0.0)
    out = lax.dot_general(h, fc_ref[...], (((1,), (1,)), ((), ())),
                          preferred_element_type=jnp.float32)
    out_ref[...] = out + fcb_ref[...]


def _tc_head(tl, W_weight, W_bias, fc_weight, fc_bias):
    return pl.pallas_call(
        _tc_head_body,
        grid=(P // BLK,),
        in_specs=[
            pl.BlockSpec((BLK, 2, D), lambda i: (i, 0, 0)),
            pl.BlockSpec((D, D), lambda i: (0, 0)),
            pl.BlockSpec((1, D), lambda i: (0, 0)),
            pl.BlockSpec((N_CLASSES, D), lambda i: (0, 0)),
            pl.BlockSpec((1, N_CLASSES), lambda i: (0, 0)),
        ],
        out_specs=pl.BlockSpec((BLK, N_CLASSES), lambda i: (i, 0)),
        out_shape=jax.ShapeDtypeStruct((P, N_CLASSES), jnp.float32),
    )(tl, W_weight, W_bias.reshape(1, D),
      fc_weight, fc_bias.reshape(1, N_CLASSES))


def kernel(token_ids, edge_src, emb_table, W_weight, W_bias, fc_weight, fc_bias):
    packed_emb = _tc_pack(emb_table)
    tok = token_ids.astype(jnp.int32)
    tok_p = jnp.pad(tok, (0, P - N_NODES)).reshape(NS, TCH, EPC)
    es = edge_src.astype(jnp.int32)
    es_p = jnp.pad(es, (0, P * DEG - N_EDGES)).reshape(NW, NCHUNK, EPC)
    tl = _sc_gather_reduce(es_p, tok_p, packed_emb)
    out = _tc_head(tl, W_weight, W_bias, fc_weight, fc_bias)
    return out[:N_NODES]
